# baseline (speedup 1.0000x reference)
import jax
import jax.numpy as jnp
from jax.experimental import pallas as pl
from jax.experimental.pallas import tpu as pltpu

B = 4096
D = 128
A = 1000


def _body(idx_ref, x_ref, w_ref, b_ref, o_ref):
    x = x_ref[0]
    w = w_ref[0]
    z = jax.lax.dot_general(x, w, (((1,), (1,)), ((), ())),
                            preferred_element_type=jnp.float32)
    o_ref[0] = z + b_ref[0]


def kernel(x, a, weight, bias):
    idx = a[:, 0].astype(jnp.int32)
    x3 = x.reshape(B, 1, D)
    b3 = bias.reshape(A, 1, D)
    out = pl.pallas_call(
        _body,
        grid_spec=pltpu.PrefetchScalarGridSpec(
            num_scalar_prefetch=1,
            grid=(B,),
            in_specs=[
                pl.BlockSpec((1, 1, D), lambda i, idx_ref: (i, 0, 0)),
                pl.BlockSpec((1, D, D), lambda i, idx_ref: (idx_ref[i], 0, 0)),
                pl.BlockSpec((1, 1, D), lambda i, idx_ref: (idx_ref[i], 0, 0)),
            ],
            out_specs=pl.BlockSpec((1, 1, D), lambda i, idx_ref: (i, 0, 0)),
        ),
        out_shape=jax.ShapeDtypeStruct((B, 1, D), jnp.float32),
    )(idx, x3, weight, b3)
    return out.reshape(B, D)


# speedup vs baseline: 11.0823x; 11.0823x over previous
import jax
import jax.numpy as jnp
from jax.experimental import pallas as pl
from jax.experimental.pallas import tpu as pltpu

B = 4096
D = 128
A = 1000
K = 16
C = B // K


def _body(sidx_ref, perm_ref, x_ref, b_ref, *rest):
    w_refs = rest[:K]
    o_ref = rest[K]
    i = pl.program_id(0)
    for k in range(K):
        s = k * C + i
        row = perm_ref[s]
        bidx = sidx_ref[s]
        x_row = x_ref[pl.ds(row, 1), :]
        z = jax.lax.dot_general(x_row, w_refs[k][0], (((1,), (1,)), ((), ())),
                                preferred_element_type=jnp.float32)
        o_ref[pl.ds(row, 1), :] = z + b_ref[pl.ds(bidx, 1), :]


def kernel(x, a, weight, bias):
    idx = a[:, 0].astype(jnp.int32)
    sidx, perm = jax.lax.sort_key_val(idx, jnp.arange(B, dtype=jnp.int32))

    def w_spec(k):
        return pl.BlockSpec(
            (1, D, D),
            lambda i, sidx_ref, perm_ref, _k=k: (sidx_ref[_k * C + i], 0, 0))

    out = pl.pallas_call(
        _body,
        grid_spec=pltpu.PrefetchScalarGridSpec(
            num_scalar_prefetch=2,
            grid=(C,),
            in_specs=[
                pl.BlockSpec((B, D), lambda i, s_, p_: (0, 0)),
                pl.BlockSpec((A, D), lambda i, s_, p_: (0, 0)),
            ] + [w_spec(k) for k in range(K)],
            out_specs=pl.BlockSpec((B, D), lambda i, s_, p_: (0, 0)),
        ),
        out_shape=jax.ShapeDtypeStruct((B, D), jnp.float32),
    )(sidx, perm, x, bias, *([weight] * K))
    return out
